# baseline (speedup 1.0000x reference)
import jax
import jax.numpy as jnp
from jax.experimental import pallas as pl
from jax.experimental.pallas import tpu as pltpu

DIM = 512
BN = 8192


def _pool_kernel(ba_ref, bg_ref, x_ref, wa_ref, wg_ref, out_ref, acc_ref, ssum_ref):
    j = pl.program_id(1)
    nj = pl.num_programs(1)

    @pl.when(j == 0)
    def _init():
        acc_ref[...] = jnp.zeros_like(acc_ref)
        ssum_ref[0, 0] = 0.0

    x = x_ref[0]
    pa = jnp.dot(x, wa_ref[...], preferred_element_type=jnp.float32)
    pg = jnp.dot(x, wg_ref[...], preferred_element_type=jnp.float32)
    a = jnp.tanh(pa + ba_ref[0, 0])
    g = jax.nn.sigmoid(pg + bg_ref[0, 0])
    e = jnp.exp(a * g)

    acc_ref[...] += jnp.sum(e * x, axis=0, keepdims=True)
    ssum_ref[0, 0] += jnp.sum(e)

    @pl.when(j == nj - 1)
    def _finish():
        out_ref[0] = acc_ref[...] / ssum_ref[0, 0]


def kernel(x, W_a, b_a, W_g, b_g):
    B, N, D = x.shape
    nj = N // BN
    out = pl.pallas_call(
        _pool_kernel,
        grid=(B, nj),
        in_specs=[
            pl.BlockSpec(memory_space=pltpu.SMEM),
            pl.BlockSpec(memory_space=pltpu.SMEM),
            pl.BlockSpec((1, BN, D), lambda b, j: (b, j, 0)),
            pl.BlockSpec((D, 1), lambda b, j: (0, 0)),
            pl.BlockSpec((D, 1), lambda b, j: (0, 0)),
        ],
        out_specs=pl.BlockSpec((1, 1, D), lambda b, j: (b, 0, 0)),
        out_shape=jax.ShapeDtypeStruct((B, 1, D), jnp.float32),
        scratch_shapes=[
            pltpu.VMEM((1, D), jnp.float32),
            pltpu.SMEM((1, 1), jnp.float32),
        ],
        compiler_params=pltpu.CompilerParams(
            dimension_semantics=("parallel", "arbitrary"),
        ),
    )(b_a.reshape(1, 1), b_g.reshape(1, 1), x, W_a, W_g)
    return out


# speedup vs baseline: 1.1661x; 1.1661x over previous
import jax
import jax.numpy as jnp
from jax.experimental import pallas as pl
from jax.experimental.pallas import tpu as pltpu

DIM = 512
BN = 8192


def _pool_kernel(b_ref, x_ref, w_ref, out_ref, acc_ref, ssum_ref):
    j = pl.program_id(1)
    nj = pl.num_programs(1)

    @pl.when(j == 0)
    def _init():
        acc_ref[...] = jnp.zeros_like(acc_ref)
        ssum_ref[0, 0] = 0.0

    x = x_ref[0]
    proj = jnp.dot(x, w_ref[...], preferred_element_type=jnp.float32)
    a = jnp.tanh(proj[:, 0:1] + b_ref[0, 0])
    g = jax.nn.sigmoid(proj[:, 1:2] + b_ref[0, 1])
    e = jnp.exp(a * g)

    acc_ref[...] += jnp.sum(e * x, axis=0, keepdims=True)
    ssum_ref[0, 0] += jnp.sum(e)

    @pl.when(j == nj - 1)
    def _finish():
        out_ref[0] = acc_ref[...] / ssum_ref[0, 0]


def kernel(x, W_a, b_a, W_g, b_g):
    B, N, D = x.shape
    w = jnp.concatenate([W_a, W_g], axis=1)
    biases = jnp.stack([b_a[0], b_g[0]]).reshape(1, 2)

    nj = N // BN
    out = pl.pallas_call(
        _pool_kernel,
        grid=(B, nj),
        in_specs=[
            pl.BlockSpec(memory_space=pltpu.SMEM),
            pl.BlockSpec((1, BN, D), lambda b, j: (b, j, 0)),
            pl.BlockSpec((D, 2), lambda b, j: (0, 0)),
        ],
        out_specs=pl.BlockSpec((1, 1, D), lambda b, j: (b, 0, 0)),
        out_shape=jax.ShapeDtypeStruct((B, 1, D), jnp.float32),
        scratch_shapes=[
            pltpu.VMEM((1, D), jnp.float32),
            pltpu.SMEM((1, 1), jnp.float32),
        ],
        compiler_params=pltpu.CompilerParams(
            dimension_semantics=("parallel", "arbitrary"),
        ),
    )(biases, x, w)
    return out
